# Initial kernel scaffold; baseline (speedup 1.0000x reference)
import jax
import jax.numpy as jnp
from jax.experimental import pallas as pl


def kernel(x, edge_index, W1, b1, W2, b2, Wc, bc):
    raise NotImplementedError("write your pallas kernel here")


# speedup vs baseline: 8.2269x; 8.2269x over previous
import functools

import jax
import jax.numpy as jnp
from jax import lax
from jax.experimental import pallas as pl
from jax.experimental.pallas import tpu as pltpu
from jax.experimental.pallas import tpu_sc as plsc

N = 10000
D = 128
D_OUT = 64
E = 320000

NC = 2
NS = 16
NW = NC * NS
CH = 128
NCHUNK = 80
EPW = NCHUNK * CH
E_PAD = NW * EPW
N_PAD = 10240
RPT = N_PAD // NS


def _mesh():
    return plsc.VectorSubcoreMesh(
        core_axis_name="c", subcore_axis_name="s", num_cores=NC, num_subcores=NS
    )


def _make_prop(width):
    @functools.partial(
        pl.kernel,
        out_type=jax.ShapeDtypeStruct((NC, N_PAD, width), jnp.float32),
        mesh=_mesh(),
        scratch_types=[
            pltpu.VMEM((NCHUNK, CH), jnp.int32),
            pltpu.VMEM((NCHUNK, CH), jnp.int32),
            pltpu.VMEM((CH, width), jnp.float32),
            pltpu.VMEM_SHARED((N_PAD, width), jnp.float32),
            pltpu.SemaphoreType.DMA,
        ],
    )
    def prop(table, gidx, sidx, zeros, out, gidx_v, sidx_v, buf, acc, sem):
        c = lax.axis_index("c")
        s = lax.axis_index("s")
        w = c * NS + s
        pltpu.sync_copy(gidx.at[w], gidx_v)
        pltpu.sync_copy(sidx.at[w], sidx_v)
        rb = s * RPT
        pltpu.sync_copy(zeros.at[pl.ds(rb, RPT)], acc.at[pl.ds(rb, RPT)])
        plsc.subcore_barrier()

        def body(j, carry):
            pltpu.async_copy(table.at[gidx_v.at[j]], buf, sem).wait()
            pltpu.sync_copy(buf, acc.at[sidx_v.at[j]], add=True)
            return carry

        lax.fori_loop(0, NCHUNK, body, 0)
        plsc.subcore_barrier()
        pltpu.sync_copy(acc.at[pl.ds(rb, RPT)], out.at[c, pl.ds(rb, RPT)])

    return prop


def _make_deg(width=128):
    @functools.partial(
        pl.kernel,
        out_type=jax.ShapeDtypeStruct((NC, N_PAD, width), jnp.float32),
        mesh=_mesh(),
        scratch_types=[
            pltpu.VMEM((NCHUNK, CH), jnp.int32),
            pltpu.VMEM((CH, width), jnp.float32),
            pltpu.VMEM_SHARED((N_PAD, width), jnp.float32),
        ],
    )
    def deg(ones, sidx, zeros, out, sidx_v, ones_v, acc):
        c = lax.axis_index("c")
        s = lax.axis_index("s")
        w = c * NS + s
        pltpu.sync_copy(sidx.at[w], sidx_v)
        pltpu.sync_copy(ones, ones_v)
        rb = s * RPT
        pltpu.sync_copy(zeros.at[pl.ds(rb, RPT)], acc.at[pl.ds(rb, RPT)])
        plsc.subcore_barrier()

        def body(j, carry):
            pltpu.sync_copy(ones_v, acc.at[sidx_v.at[j]], add=True)
            return carry

        lax.fori_loop(0, NCHUNK, body, 0)
        plsc.subcore_barrier()
        pltpu.sync_copy(acc.at[pl.ds(rb, RPT)], out.at[c, pl.ds(rb, RPT)])

    return deg


_GRID = 16
_BR = N_PAD // _GRID


def _dinv_block(d0, d1):
    deg = d0[:, 0:1] + d1[:, 0:1] + 1.0
    return lax.rsqrt(deg)


def _tc_first(xp, w1, d0, d1):
    def body(x_ref, w_ref, d0_ref, d1_ref, o_ref):
        dinv = _dinv_block(d0_ref[...], d1_ref[...])
        h = lax.dot_general(
            x_ref[...], w_ref[...], (((1,), (1,)), ((), ())),
            preferred_element_type=jnp.float32,
        )
        o_ref[...] = h * dinv

    return pl.pallas_call(
        body,
        grid=(_GRID,),
        in_specs=[
            pl.BlockSpec((_BR, D), lambda i: (i, 0)),
            pl.BlockSpec((D, D), lambda i: (0, 0)),
            pl.BlockSpec((_BR, D), lambda i: (i, 0)),
            pl.BlockSpec((_BR, D), lambda i: (i, 0)),
        ],
        out_specs=pl.BlockSpec((_BR, D), lambda i: (i, 0)),
        out_shape=jax.ShapeDtypeStruct((N_PAD, D), jnp.float32),
    )(xp, w1, d0, d1)


def _tc_mid(s0, s1, u1, w2, b1, d0, d1):
    def body(s0_ref, s1_ref, u_ref, w_ref, b_ref, d0_ref, d1_ref, o_ref):
        dinv = _dinv_block(d0_ref[...], d1_ref[...])
        agg = s0_ref[...] + s1_ref[...] + u_ref[...]
        h1 = jnp.maximum(agg * dinv + b_ref[...], 0.0)
        h2 = lax.dot_general(
            h1, w_ref[...], (((1,), (1,)), ((), ())),
            preferred_element_type=jnp.float32,
        )
        o_ref[...] = h2 * dinv

    return pl.pallas_call(
        body,
        grid=(_GRID,),
        in_specs=[
            pl.BlockSpec((_BR, D), lambda i: (i, 0)),
            pl.BlockSpec((_BR, D), lambda i: (i, 0)),
            pl.BlockSpec((_BR, D), lambda i: (i, 0)),
            pl.BlockSpec((D, D), lambda i: (0, 0)),
            pl.BlockSpec((1, D), lambda i: (0, 0)),
            pl.BlockSpec((_BR, D), lambda i: (i, 0)),
            pl.BlockSpec((_BR, D), lambda i: (i, 0)),
        ],
        out_specs=pl.BlockSpec((_BR, D), lambda i: (i, 0)),
        out_shape=jax.ShapeDtypeStruct((N_PAD, D), jnp.float32),
    )(s0, s1, u1, w2, b1, d0, d1)


def _tc_last(s0, s1, u2, wc, b2, bc, d0, d1):
    def body(s0_ref, s1_ref, u_ref, w_ref, b2_ref, bc_ref, d0_ref, d1_ref, o_ref):
        dinv = _dinv_block(d0_ref[...], d1_ref[...])
        agg = s0_ref[...] + s1_ref[...] + u_ref[...]
        h2 = agg * dinv + b2_ref[...]
        o = lax.dot_general(
            h2, w_ref[...], (((1,), (1,)), ((), ())),
            preferred_element_type=jnp.float32,
        )
        o_ref[...] = o + bc_ref[...]

    return pl.pallas_call(
        body,
        grid=(_GRID,),
        in_specs=[
            pl.BlockSpec((_BR, D), lambda i: (i, 0)),
            pl.BlockSpec((_BR, D), lambda i: (i, 0)),
            pl.BlockSpec((_BR, D), lambda i: (i, 0)),
            pl.BlockSpec((D_OUT, D), lambda i: (0, 0)),
            pl.BlockSpec((1, D), lambda i: (0, 0)),
            pl.BlockSpec((1, D_OUT), lambda i: (0, 0)),
            pl.BlockSpec((_BR, D), lambda i: (i, 0)),
            pl.BlockSpec((_BR, D), lambda i: (i, 0)),
        ],
        out_specs=pl.BlockSpec((_BR, D_OUT), lambda i: (i, 0)),
        out_shape=jax.ShapeDtypeStruct((N, D_OUT), jnp.float32),
    )(s0, s1, u2, wc, b2, bc, d0, d1)


def kernel(x, edge_index, W1, b1, W2, b2, Wc, bc):
    src = edge_index[0]
    dst = edge_index[1]
    pad = jnp.full((E_PAD - E,), N, dtype=jnp.int32)
    srcp = jnp.concatenate([src, pad]).reshape(NW, NCHUNK, CH)
    dstp = jnp.concatenate([dst, pad]).reshape(NW, NCHUNK, CH)

    xp = jnp.zeros((N_PAD, D), jnp.float32).at[:N].set(x)
    ones128 = jnp.ones((CH, D), jnp.float32)
    zeros128 = jnp.zeros((N_PAD, D), jnp.float32)
    b1r = b1.reshape(1, D)
    b2r = b2.reshape(1, D)
    bcr = bc.reshape(1, D_OUT)

    deg = _make_deg(D)(ones128, dstp, zeros128)
    d0, d1 = deg[0], deg[1]

    prop = _make_prop(D)
    u1 = _tc_first(xp, W1, d0, d1)
    s1 = prop(u1, srcp, dstp, zeros128)
    u2 = _tc_mid(s1[0], s1[1], u1, W2, b1r, d0, d1)
    s2 = prop(u2, srcp, dstp, zeros128)
    return _tc_last(s2[0], s2[1], u2, Wc, b2r, bcr, d0, d1)
